# baseline (speedup 1.0000x reference)
import functools

import jax
import jax.numpy as jnp
from jax import lax
from jax.experimental import pallas as pl
from jax.experimental.pallas import tpu as pltpu
from jax.experimental.pallas import tpu_sc as plsc

L = 16


@functools.lru_cache(maxsize=None)
def _build(B, T, E):
    W = 2 * E + 1
    info = plsc.get_sparse_core_info()
    NC, NS = info.num_cores, info.num_subcores
    NW = NC * NS
    BPW = B // NW
    GROUPS = BPW // L
    mesh = plsc.VectorSubcoreMesh(core_axis_name="c", subcore_axis_name="s")

    @functools.partial(
        pl.kernel,
        mesh=mesh,
        out_type=jax.ShapeDtypeStruct((B,), jnp.float32),
        compiler_params=pltpu.CompilerParams(needs_layout_passes=False),
        scratch_types=[
            pltpu.VMEM((2 * T + L,), jnp.float32),
            pltpu.VMEM((2 * BPW,), jnp.float32),
            pltpu.VMEM((2 * BPW,), jnp.int32),
            pltpu.VMEM((2 * BPW,), jnp.float32),
            pltpu.SemaphoreType.DMA,
            pltpu.SemaphoreType.DMA,
            pltpu.SemaphoreType.DMA,
        ],
    )
    def hwnet_sc(tab_hbm, tc_hbm, x_hbm, out_hbm, evvec, xo,
                 ints, flts, sem_t, sem_v, sem_x):
        wid = lax.axis_index("s") * NC + lax.axis_index("c")
        base = wid * BPW
        h_e = pltpu.async_copy(tab_hbm.at[pl.ds(0, T)],
                               evvec.at[pl.ds(0, T)], sem_t)
        h_v = pltpu.async_copy(tab_hbm.at[pl.ds(T, T)],
                               evvec.at[pl.ds(T, T)], sem_v)
        h_x = pltpu.async_copy(x_hbm.at[pl.ds(base, BPW)],
                               xo.at[pl.ds(0, BPW)], sem_x)
        h_x.wait()
        h_e.wait()

        LOG = T.bit_length() - 1

        def lower_bound_n(keys):
            n = len(keys)

            def body(i, carry):
                ps = carry[:n]
                half = carry[n]
                hm1 = half - 1
                vs = [plsc.load_gather(evvec, [p + hm1]) for p in ps]
                ps = [jnp.where(v < k, p + half, p)
                      for v, k, p in zip(vs, keys, ps)]
                return (*ps, half >> 1)

            z = jnp.zeros((L,), jnp.int32)
            h0 = jnp.full((L,), T // 2, jnp.int32)
            out = lax.fori_loop(0, LOG, body, (z,) * n + (h0,))
            return out[:n]

        def nearest(x, pos):
            a = jnp.maximum(pos - 1, 0)
            ea = plsc.load_gather(evvec, [a])
            eb = plsc.load_gather(evvec, [pos])
            ra = x - ea
            rb = x - eb
            da = ra * ra
            db = rb * rb
            take_a = da <= db
            vstar = jnp.where(take_a, ea, eb)
            dmin = jnp.minimum(da, db)
            cand = jnp.where(take_a, a, pos)
            return vstar, dmin, cand

        SW = 4

        def search_quad(jj, _):
            j0 = jj * SW
            xs = [xo[pl.ds((j0 + k) * L, L)] for k in range(SW)]
            poss = lower_bound_n(xs)
            vdc = [nearest(x, p) for x, p in zip(xs, poss)]
            vs = [t[0] for t in vdc]
            dmins = [t[1] for t in vdc]
            cs = [t[2] for t in vdc]
            pas = [plsc.load_gather(evvec, [jnp.maximum(c - 1, 0)])
                   for c in cs]
            dups = [(pa == v) & (c > 0) for pa, v, c in zip(pas, vs, cs)]
            any_dup = functools.reduce(
                lambda a, b: a | b, [jnp.any(d) for d in dups])

            def slow(_):
                fs = lower_bound_n(vs)
                return tuple(jnp.where(d, f, c)
                             for d, f, c in zip(dups, fs, cs))

            idxs = lax.cond(any_dup, slow, lambda _: tuple(cs), 0)

            for k in range(SW):
                o = (j0 + k) * L
                ints[pl.ds(o, L)] = idxs[k]
                ints[pl.ds(BPW + o, L)] = (
                    jnp.clip(idxs[k], E, T - E - 1) - E)
                flts[pl.ds(o, L)] = dmins[k]
            return 0

        lax.fori_loop(0, GROUPS // SW, search_quad, 0)

        pltpu.async_copy(tc_hbm.at[ints.at[pl.ds(0, BPW)]],
                         flts.at[pl.ds(BPW, BPW)], sem_x).wait()
        h_v.wait()

        IOTA = lax.iota(jnp.int32, L)
        CHUNKS = (W + L - 1) // L
        zero = jnp.zeros((L,), jnp.float32)

        def bcast(v, li):
            return jnp.take_along_axis(v, li, axis=0,
                                       mode="promise_in_bounds")

        def window_group(j, _):
            o = j * L
            x = xo[pl.ds(o, L)]
            s = ints[pl.ds(BPW + o, L)]
            dmin = flts[pl.ds(o, L)]
            tc = flts[pl.ds(BPW + o, L)]

            @plsc.parallel_loop(0, L, 1, unroll=2, carry=(zero, zero))
            def oacc(lane, carry):
                onum, oden = carry
                li = jnp.full((L,), lane, jnp.int32)
                xb = bcast(x, li)
                tb = bcast(tc, li)
                cb = tb * bcast(dmin, li)
                idx0 = bcast(s, li) + IOTA
                nsum = zero
                dsum = zero
                for r in range(CHUNKS):
                    idx = idx0 + (r * L)
                    ew = plsc.load_gather(evvec, [idx])
                    vw = plsc.load_gather(evvec, [idx + T])
                    rr = xb - ew
                    e = jnp.exp(cb - tb * (rr * rr))
                    if r == CHUNKS - 1:
                        m = IOTA < (W - L * r)
                        e = jnp.where(m, e, 0.0)
                        vw = jnp.where(m, vw, 0.0)
                    nsum = nsum + vw * e
                    dsum = dsum + e
                sel = IOTA == lane
                onum = jnp.where(sel, jnp.sum(nsum), onum)
                oden = jnp.where(sel, jnp.sum(dsum), oden)
                return (onum, oden)

            onum, oden = oacc
            xo[pl.ds(BPW + o, L)] = onum / oden
            return 0

        lax.fori_loop(0, GROUPS, window_group, 0)
        pltpu.sync_copy(xo.at[pl.ds(BPW, BPW)],
                        out_hbm.at[pl.ds(base, BPW)])

    return hwnet_sc


def kernel(inputs, evaluate_table, takecare_table, vector_table, idx_table):
    B = inputs.shape[0]
    T = evaluate_table.shape[0]
    E = (idx_table.shape[0] - 1) // 2
    D = vector_table.shape[1]
    assert D == 1
    tables = jnp.concatenate(
        [evaluate_table.reshape(T), vector_table.reshape(T)])
    fn = _build(B, T, E)
    out = fn(tables, takecare_table.reshape(T), inputs.reshape(B))
    return out.reshape(B, D)


# speedup vs baseline: 1.0124x; 1.0124x over previous
import functools

import jax
import jax.numpy as jnp
from jax import lax
from jax.experimental import pallas as pl
from jax.experimental.pallas import tpu as pltpu
from jax.experimental.pallas import tpu_sc as plsc

L = 16


@functools.lru_cache(maxsize=None)
def _build(B, T, E):
    W = 2 * E + 1
    info = plsc.get_sparse_core_info()
    NC, NS = info.num_cores, info.num_subcores
    NW = NC * NS
    BPW = B // NW
    GROUPS = BPW // L
    mesh = plsc.VectorSubcoreMesh(core_axis_name="c", subcore_axis_name="s")

    @functools.partial(
        pl.kernel,
        mesh=mesh,
        out_type=jax.ShapeDtypeStruct((B,), jnp.float32),
        compiler_params=pltpu.CompilerParams(needs_layout_passes=False),
        scratch_types=[
            pltpu.VMEM((2 * T,), jnp.float32),
            pltpu.VMEM((2 * BPW,), jnp.float32),
            pltpu.VMEM((2 * BPW,), jnp.int32),
            pltpu.VMEM((2 * BPW,), jnp.float32),
            pltpu.SemaphoreType.DMA,
            pltpu.SemaphoreType.DMA,
            pltpu.SemaphoreType.DMA,
        ],
    )
    def hwnet_sc(tab_hbm, tc_hbm, x_hbm, out_hbm, evvec, xo,
                 ints, flts, sem_t, sem_v, sem_x):
        wid = lax.axis_index("s") * NC + lax.axis_index("c")
        base = wid * BPW
        h_e = pltpu.async_copy(tab_hbm.at[pl.ds(0, T)],
                               evvec.at[pl.ds(0, T)], sem_t)
        h_v = pltpu.async_copy(tab_hbm.at[pl.ds(T, T)],
                               evvec.at[pl.ds(T, T)], sem_v)
        h_x = pltpu.async_copy(x_hbm.at[pl.ds(base, BPW)],
                               xo.at[pl.ds(0, BPW)], sem_x)
        h_x.wait()
        h_e.wait()

        LOG = T.bit_length() - 1

        def lower_bound_n(keys):
            n = len(keys)

            def body(i, carry):
                ps = carry[:n]
                half = carry[n]
                hm1 = half - 1
                vs = [plsc.load_gather(evvec, [p + hm1]) for p in ps]
                ps = [jnp.where(v < k, p + half, p)
                      for v, k, p in zip(vs, keys, ps)]
                return (*ps, half >> 1)

            z = jnp.zeros((L,), jnp.int32)
            h0 = jnp.full((L,), T // 2, jnp.int32)
            out = lax.fori_loop(0, LOG, body, (z,) * n + (h0,))
            return out[:n]

        def nearest(x, pos):
            a = jnp.maximum(pos - 1, 0)
            ea = plsc.load_gather(evvec, [a])
            eb = plsc.load_gather(evvec, [pos])
            ra = x - ea
            rb = x - eb
            da = ra * ra
            db = rb * rb
            take_a = da <= db
            vstar = jnp.where(take_a, ea, eb)
            dmin = jnp.minimum(da, db)
            cand = jnp.where(take_a, a, pos)
            return vstar, dmin, cand

        SW = 4

        def search_quad(jj, _):
            j0 = jj * SW
            xs = [xo[pl.ds((j0 + k) * L, L)] for k in range(SW)]
            poss = lower_bound_n(xs)
            vdc = [nearest(x, p) for x, p in zip(xs, poss)]
            vs = [t[0] for t in vdc]
            dmins = [t[1] for t in vdc]
            cs = [t[2] for t in vdc]
            pas = [plsc.load_gather(evvec, [jnp.maximum(c - 1, 0)])
                   for c in cs]
            dups = [(pa == v) & (c > 0) for pa, v, c in zip(pas, vs, cs)]
            any_dup = functools.reduce(
                lambda a, b: a | b, [jnp.any(d) for d in dups])

            def slow(_):
                fs = lower_bound_n(vs)
                return tuple(jnp.where(d, f, c)
                             for d, f, c in zip(dups, fs, cs))

            idxs = lax.cond(any_dup, slow, lambda _: tuple(cs), 0)

            for k in range(SW):
                o = (j0 + k) * L
                ints[pl.ds(o, L)] = idxs[k]
                ints[pl.ds(BPW + o, L)] = (
                    jnp.clip(idxs[k], E, T - E - 1) - E)
                flts[pl.ds(o, L)] = dmins[k]
            return 0

        lax.fori_loop(0, GROUPS // SW, search_quad, 0)

        pltpu.async_copy(tc_hbm.at[ints.at[pl.ds(0, BPW)]],
                         flts.at[pl.ds(BPW, BPW)], sem_x).wait()
        h_v.wait()

        def window_pair(jj, _):
            j0 = jj * 2
            o0 = j0 * L
            o1 = j0 * L + L
            x0 = xo[pl.ds(o0, L)]
            x1 = xo[pl.ds(o1, L)]
            s0 = ints[pl.ds(BPW + o0, L)]
            s1 = ints[pl.ds(BPW + o1, L)]
            dmin0 = flts[pl.ds(o0, L)]
            dmin1 = flts[pl.ds(o1, L)]
            tc0 = flts[pl.ds(BPW + o0, L)]
            tc1 = flts[pl.ds(BPW + o1, L)]

            zero = jnp.zeros((L,), jnp.float32)

            @plsc.parallel_loop(0, W, 1, unroll=8,
                                carry=(zero, zero, zero, zero))
            def wresult(w, carry):
                n0, d0, n1, d1 = carry
                i0 = s0 + w
                i1 = s1 + w
                ew0 = plsc.load_gather(evvec, [i0])
                vw0 = plsc.load_gather(evvec, [i0 + T])
                ew1 = plsc.load_gather(evvec, [i1])
                vw1 = plsc.load_gather(evvec, [i1 + T])
                r0 = x0 - ew0
                r1 = x1 - ew1
                e0 = jnp.exp(tc0 * (dmin0 - r0 * r0))
                e1 = jnp.exp(tc1 * (dmin1 - r1 * r1))
                return (n0 + vw0 * e0, d0 + e0, n1 + vw1 * e1, d1 + e1)

            n0, d0, n1, d1 = wresult
            xo[pl.ds(BPW + o0, L)] = n0 / d0
            xo[pl.ds(BPW + o1, L)] = n1 / d1
            return 0

        lax.fori_loop(0, GROUPS // 2, window_pair, 0)
        pltpu.sync_copy(xo.at[pl.ds(BPW, BPW)],
                        out_hbm.at[pl.ds(base, BPW)])

    return hwnet_sc


def kernel(inputs, evaluate_table, takecare_table, vector_table, idx_table):
    B = inputs.shape[0]
    T = evaluate_table.shape[0]
    E = (idx_table.shape[0] - 1) // 2
    D = vector_table.shape[1]
    assert D == 1
    tables = jnp.concatenate(
        [evaluate_table.reshape(T), vector_table.reshape(T)])
    fn = _build(B, T, E)
    out = fn(tables, takecare_table.reshape(T), inputs.reshape(B))
    return out.reshape(B, D)
